# Initial kernel scaffold; baseline (speedup 1.0000x reference)
import functools

import jax
import jax.numpy as jnp
from jax import lax
from jax.experimental import pallas as pl
from jax.experimental.pallas import tpu as pltpu
from jax.experimental.pallas import tpu_sc as plsc

N = 10000
NPAD = 10016
D = 16
NCLS = 64
NC = 2
NS = 16
NW = NC * NS
E = 320000
CHUNK = 128
CPW = (E + NW * CHUNK - 1) // (NW * CHUNK)
EPW = CPW * CHUNK
EPAD = EPW * NW
ROWS_PT = NPAD // NS

_mesh = plsc.VectorSubcoreMesh(core_axis_name="c", subcore_axis_name="s")


def _deg_body(dst_hbm, ones_hbm, zeros_hbm, out_hbm, idxv, onesv, degsh, sem):
    cid = lax.axis_index("c")
    sid = lax.axis_index("s")
    wid = sid * NC + cid

    pltpu.sync_copy(zeros_hbm.at[pl.ds(sid * ROWS_PT, ROWS_PT)],
                    degsh.at[pl.ds(sid * ROWS_PT, ROWS_PT)])
    pltpu.sync_copy(ones_hbm.at[pl.ds(0, CHUNK)], onesv)
    pltpu.sync_copy(dst_hbm.at[wid], idxv)
    plsc.subcore_barrier()

    def body(c, carry):
        pltpu.sync_copy(onesv, degsh.at[idxv.at[c]], add=True)
        return carry

    lax.fori_loop(0, CPW, body, 0)
    plsc.subcore_barrier()
    pltpu.sync_copy(degsh.at[pl.ds(sid * ROWS_PT, ROWS_PT)],
                    out_hbm.at[cid, pl.ds(sid * ROWS_PT, ROWS_PT)])


_deg_call = functools.partial(
    pl.kernel, _deg_body, mesh=_mesh,
    out_type=jax.ShapeDtypeStruct((NC, NPAD), jnp.float32),
    scratch_types=[
        pltpu.VMEM((CPW, CHUNK), jnp.int32),
        pltpu.VMEM((CHUNK,), jnp.float32),
        pltpu.VMEM_SHARED((NPAD,), jnp.float32),
        pltpu.SemaphoreType.DMA,
    ],
)()


def _edge_body(ht_hbm, src_hbm, dst_hbm, zeros_hbm, out_hbm,
               srcv, dstv, rows0, rows1, accsh, sem0, sem1):
    cid = lax.axis_index("c")
    sid = lax.axis_index("s")
    wid = sid * NC + cid

    pltpu.sync_copy(zeros_hbm.at[pl.ds(sid * ROWS_PT, ROWS_PT)],
                    accsh.at[pl.ds(sid * ROWS_PT, ROWS_PT)])
    pltpu.sync_copy(src_hbm.at[wid], srcv)
    pltpu.sync_copy(dst_hbm.at[wid], dstv)
    plsc.subcore_barrier()

    def pair_body(p, carry):
        c0 = p * 2
        c1 = c0 + 1
        d0 = pltpu.async_copy(ht_hbm.at[srcv.at[c0]], rows0, sem0)
        d1 = pltpu.async_copy(ht_hbm.at[srcv.at[c1]], rows1, sem1)
        d0.wait()
        pltpu.sync_copy(rows0, accsh.at[dstv.at[c0]], add=True)
        d1.wait()
        pltpu.sync_copy(rows1, accsh.at[dstv.at[c1]], add=True)
        return carry

    lax.fori_loop(0, CPW // 2, pair_body, 0)
    pltpu.async_copy(ht_hbm.at[srcv.at[CPW - 1]], rows0, sem0).wait()
    pltpu.sync_copy(rows0, accsh.at[dstv.at[CPW - 1]], add=True)

    plsc.subcore_barrier()
    pltpu.sync_copy(accsh.at[pl.ds(sid * ROWS_PT, ROWS_PT)],
                    out_hbm.at[cid, pl.ds(sid * ROWS_PT, ROWS_PT)])


_edge_call = functools.partial(
    pl.kernel, _edge_body, mesh=_mesh,
    out_type=jax.ShapeDtypeStruct((NC, NPAD, D), jnp.float32),
    scratch_types=[
        pltpu.VMEM((CPW, CHUNK), jnp.int32),
        pltpu.VMEM((CPW, CHUNK), jnp.int32),
        pltpu.VMEM((CHUNK, D), jnp.float32),
        pltpu.VMEM((CHUNK, D), jnp.float32),
        pltpu.VMEM_SHARED((NPAD, D), jnp.float32),
        pltpu.SemaphoreType.DMA,
        pltpu.SemaphoreType.DMA,
    ],
)()


def _prep_body(x_ref, w_ref, b_ref, d2_ref, ht_ref, dv_ref):
    deg = jnp.maximum(d2_ref[0] + d2_ref[1] + 1.0, 1.0)
    dinv = lax.rsqrt(deg)
    dv = jnp.broadcast_to(dinv[:, None], (NPAD, D))
    dv_ref[...] = dv
    hw = jnp.dot(x_ref[...], w_ref[...],
                 preferred_element_type=jnp.float32) + b_ref[...]
    ht_ref[...] = hw * dv


def _prep_call(xp, W0, b0, deg2):
    return pl.pallas_call(
        _prep_body,
        out_shape=[jax.ShapeDtypeStruct((NPAD, D), jnp.float32),
                   jax.ShapeDtypeStruct((NPAD, D), jnp.float32)],
    )(xp, W0, b0, deg2)


def _mid_body(acc_ref, htp_ref, dv_ref, m_ref, w_ref, b_ref, ht_ref, mo_ref):
    dv = dv_ref[...]
    h = jnp.maximum(dv * (acc_ref[0] + acc_ref[1] + htp_ref[...]), 0.0)
    mo_ref[...] = jnp.maximum(m_ref[...], h)
    ht_ref[...] = (jnp.dot(h, w_ref[...],
                           preferred_element_type=jnp.float32) + b_ref[...]) * dv


def _mid_call(acc, htp, dv, m, W, b):
    return pl.pallas_call(
        _mid_body,
        out_shape=[jax.ShapeDtypeStruct((NPAD, D), jnp.float32),
                   jax.ShapeDtypeStruct((NPAD, D), jnp.float32)],
    )(acc, htp, dv, m, W, b)


def _final_body(acc_ref, htp_ref, dv_ref, m_ref, w_ref, b_ref, out_ref):
    h = jnp.maximum(dv_ref[...] * (acc_ref[0] + acc_ref[1] + htp_ref[...]), 0.0)
    m = jnp.maximum(m_ref[...], h)
    o = jnp.dot(m, w_ref[...], preferred_element_type=jnp.float32) + b_ref[...]
    o = o - jnp.max(o, axis=1, keepdims=True)
    out_ref[...] = o - jnp.log(jnp.sum(jnp.exp(o), axis=1, keepdims=True))


def _final_call(acc, htp, dv, m, fcW, fcb):
    return pl.pallas_call(
        _final_body,
        out_shape=jax.ShapeDtypeStruct((NPAD, NCLS), jnp.float32),
    )(acc, htp, dv, m, fcW, fcb)


def kernel(x, edge_index, W0, b0, W1, b1, W2, b2, W3, b3, W4, b4, W5, b5,
           fcW, fcb):
    pad = (jnp.arange(EPAD - E, dtype=jnp.int32) % (NPAD - N)) + N
    srcp = jnp.concatenate([edge_index[0], pad]).reshape(NW, CPW, CHUNK)
    dstp = jnp.concatenate([edge_index[1], pad]).reshape(NW, CPW, CHUNK)
    xp = jnp.pad(x, ((0, NPAD - N), (0, 0)))
    zeros2 = jnp.zeros((NPAD, D), jnp.float32)
    zeros1 = jnp.zeros((NPAD,), jnp.float32)
    ones1 = jnp.ones((NPAD,), jnp.float32)

    deg2 = _deg_call(dstp, ones1, zeros1)
    ht, dv = _prep_call(xp, W0, b0.reshape(1, D), deg2)
    m = jnp.zeros((NPAD, D), jnp.float32)
    for W, b in ((W1, b1), (W2, b2), (W3, b3), (W4, b4), (W5, b5)):
        acc = _edge_call(ht, srcp, dstp, zeros2)
        ht, m = _mid_call(acc, ht, dv, m, W, b.reshape(1, D))
    acc = _edge_call(ht, srcp, dstp, zeros2)
    out = _final_call(acc, ht, dv, m, fcW, fcb.reshape(1, NCLS))
    return out[:N]


# speedup vs baseline: 34.5294x; 34.5294x over previous
import functools

import jax
import jax.numpy as jnp
from jax import lax
from jax.experimental import pallas as pl
from jax.experimental.pallas import tpu as pltpu
from jax.experimental.pallas import tpu_sc as plsc

N = 10000
NPAD = 10240
D = 16
NCLS = 64
NC = 2
NS = 16
NW = NC * NS
E = 320000
CHUNK = 128
CPW = (E + NW * CHUNK - 1) // (NW * CHUNK)
EPW = CPW * CHUNK
EPAD = EPW * NW
ROWS_PT = NPAD // NS


def _deg_body(dst_hbm, ones_hbm, zeros_hbm, out_hbm, idxv, onesv, degsh, sem):
    cid = lax.axis_index("c")
    sid = lax.axis_index("s")
    wid = sid * NC + cid

    pltpu.sync_copy(zeros_hbm.at[pl.ds(sid * ROWS_PT, ROWS_PT)],
                    degsh.at[pl.ds(sid * ROWS_PT, ROWS_PT)])
    pltpu.sync_copy(ones_hbm.at[pl.ds(0, CHUNK)], onesv)
    pltpu.sync_copy(dst_hbm.at[wid], idxv)
    plsc.subcore_barrier()

    def body(c, carry):
        pltpu.sync_copy(onesv, degsh.at[idxv.at[c]], add=True)
        return carry

    lax.fori_loop(0, CPW, body, 0)
    plsc.subcore_barrier()
    pltpu.sync_copy(degsh.at[pl.ds(sid * ROWS_PT, ROWS_PT)],
                    out_hbm.at[cid, pl.ds(sid * ROWS_PT, ROWS_PT)])


@functools.cache
def _deg_call():
    mesh = plsc.VectorSubcoreMesh(core_axis_name="c", subcore_axis_name="s",
                                  num_cores=NC, num_subcores=NS)
    return pl.kernel(
        _deg_body, mesh=mesh,
        compiler_params=pltpu.CompilerParams(use_tc_tiling_on_sc=False),
        out_type=jax.ShapeDtypeStruct((NC, NPAD), jnp.float32),
        scratch_types=[
            pltpu.VMEM((CPW, CHUNK), jnp.int32),
            pltpu.VMEM((CHUNK,), jnp.float32),
            pltpu.VMEM_SHARED((NPAD,), jnp.float32),
            pltpu.SemaphoreType.DMA,
        ],
    )


def _edge_body(ht_hbm, src_hbm, dst_hbm, zeros_hbm, out_hbm,
               srcv, dstv, rows0, rows1, accsh, sem0, sem1):
    cid = lax.axis_index("c")
    sid = lax.axis_index("s")
    wid = sid * NC + cid

    pltpu.sync_copy(zeros_hbm.at[pl.ds(sid * ROWS_PT, ROWS_PT)],
                    accsh.at[pl.ds(sid * ROWS_PT, ROWS_PT)])
    pltpu.sync_copy(src_hbm.at[wid], srcv)
    pltpu.sync_copy(dst_hbm.at[wid], dstv)
    plsc.subcore_barrier()

    def pair_body(p, carry):
        c0 = p * 2
        c1 = c0 + 1
        d0 = pltpu.async_copy(ht_hbm.at[srcv.at[c0]], rows0, sem0)
        d1 = pltpu.async_copy(ht_hbm.at[srcv.at[c1]], rows1, sem1)
        d0.wait()
        pltpu.sync_copy(rows0, accsh.at[dstv.at[c0]], add=True)
        d1.wait()
        pltpu.sync_copy(rows1, accsh.at[dstv.at[c1]], add=True)
        return carry

    lax.fori_loop(0, CPW // 2, pair_body, 0)
    pltpu.async_copy(ht_hbm.at[srcv.at[CPW - 1]], rows0, sem0).wait()
    pltpu.sync_copy(rows0, accsh.at[dstv.at[CPW - 1]], add=True)

    plsc.subcore_barrier()
    pltpu.sync_copy(accsh.at[pl.ds(sid * ROWS_PT, ROWS_PT)],
                    out_hbm.at[cid, pl.ds(sid * ROWS_PT, ROWS_PT)])


@functools.cache
def _edge_call():
    mesh = plsc.VectorSubcoreMesh(core_axis_name="c", subcore_axis_name="s",
                                  num_cores=NC, num_subcores=NS)
    return pl.kernel(
        _edge_body, mesh=mesh,
        compiler_params=pltpu.CompilerParams(use_tc_tiling_on_sc=False),
        out_type=jax.ShapeDtypeStruct((NC, NPAD, D), jnp.float32),
        scratch_types=[
            pltpu.VMEM((CPW, CHUNK), jnp.int32),
            pltpu.VMEM((CPW, CHUNK), jnp.int32),
            pltpu.VMEM((CHUNK, D), jnp.float32),
            pltpu.VMEM((CHUNK, D), jnp.float32),
            pltpu.VMEM_SHARED((NPAD, D), jnp.float32),
            pltpu.SemaphoreType.DMA,
            pltpu.SemaphoreType.DMA,
        ],
    )


def _prep_body(x_ref, w_ref, b_ref, d2_ref, ht_ref, dv_ref):
    deg = jnp.maximum(d2_ref[0] + d2_ref[1] + 1.0, 1.0)
    dinv = lax.rsqrt(deg)
    dv = jnp.broadcast_to(dinv[:, None], (NPAD, D))
    dv_ref[...] = dv
    hw = jnp.dot(x_ref[...], w_ref[...],
                 preferred_element_type=jnp.float32) + b_ref[...]
    ht_ref[...] = hw * dv


def _prep_call(xp, W0, b0, deg2):
    return pl.pallas_call(
        _prep_body,
        out_shape=[jax.ShapeDtypeStruct((NPAD, D), jnp.float32),
                   jax.ShapeDtypeStruct((NPAD, D), jnp.float32)],
    )(xp, W0, b0, deg2)


def _mid_body(acc_ref, htp_ref, dv_ref, m_ref, w_ref, b_ref, ht_ref, mo_ref):
    dv = dv_ref[...]
    h = jnp.maximum(dv * (acc_ref[0] + acc_ref[1] + htp_ref[...]), 0.0)
    mo_ref[...] = jnp.maximum(m_ref[...], h)
    ht_ref[...] = (jnp.dot(h, w_ref[...],
                           preferred_element_type=jnp.float32) + b_ref[...]) * dv


def _mid_call(acc, htp, dv, m, W, b):
    return pl.pallas_call(
        _mid_body,
        out_shape=[jax.ShapeDtypeStruct((NPAD, D), jnp.float32),
                   jax.ShapeDtypeStruct((NPAD, D), jnp.float32)],
    )(acc, htp, dv, m, W, b)


def _final_body(acc_ref, htp_ref, dv_ref, m_ref, w_ref, b_ref, out_ref):
    h = jnp.maximum(dv_ref[...] * (acc_ref[0] + acc_ref[1] + htp_ref[...]), 0.0)
    m = jnp.maximum(m_ref[...], h)
    o = jnp.dot(m, w_ref[...], preferred_element_type=jnp.float32) + b_ref[...]
    o = o - jnp.max(o, axis=1, keepdims=True)
    out_ref[...] = o - jnp.log(jnp.sum(jnp.exp(o), axis=1, keepdims=True))


def _final_call(acc, htp, dv, m, fcW, fcb):
    return pl.pallas_call(
        _final_body,
        out_shape=jax.ShapeDtypeStruct((NPAD, NCLS), jnp.float32),
    )(acc, htp, dv, m, fcW, fcb)


def kernel(x, edge_index, W0, b0, W1, b1, W2, b2, W3, b3, W4, b4, W5, b5,
           fcW, fcb):
    pad = (jnp.arange(EPAD - E, dtype=jnp.int32) % (NPAD - N)) + N
    srcp = jnp.concatenate([edge_index[0], pad]).reshape(NW, CPW, CHUNK)
    dstp = jnp.concatenate([edge_index[1], pad]).reshape(NW, CPW, CHUNK)
    xp = jnp.pad(x, ((0, NPAD - N), (0, 0)))
    zeros2 = jnp.zeros((NPAD, D), jnp.float32)
    zeros1 = jnp.zeros((NPAD,), jnp.float32)
    ones1 = jnp.ones((NPAD,), jnp.float32)

    deg2 = _deg_call()(dstp, ones1, zeros1)
    ht, dv = _prep_call(xp, W0, b0.reshape(1, D), deg2)
    m = jnp.zeros((NPAD, D), jnp.float32)
    for W, b in ((W1, b1), (W2, b2), (W3, b3), (W4, b4), (W5, b5)):
        acc = _edge_call()(ht, srcp, dstp, zeros2)
        ht, m = _mid_call(acc, ht, dv, m, W, b.reshape(1, D))
    acc = _edge_call()(ht, srcp, dstp, zeros2)
    out = _final_call(acc, ht, dv, m, fcW, fcb.reshape(1, NCLS))
    return out[:N]


# speedup vs baseline: 48.1843x; 1.3955x over previous
import functools

import jax
import jax.numpy as jnp
from jax import lax
from jax.experimental import pallas as pl
from jax.experimental.pallas import tpu as pltpu
from jax.experimental.pallas import tpu_sc as plsc

N = 10000
NPAD = 10240
D = 16
NCLS = 64
NC = 2
NS = 16
NW = NC * NS
E = 320000
CHUNK = 128
CPW = (E + NW * CHUNK - 1) // (NW * CHUNK)
EPW = CPW * CHUNK
EPAD = EPW * NW
ROWS_PT = NPAD // NS


def _deg_body(dst_hbm, ones_hbm, zeros_hbm, out_hbm, idxv, onesv, degsh, sem):
    cid = lax.axis_index("c")
    sid = lax.axis_index("s")
    wid = sid * NC + cid

    pltpu.sync_copy(zeros_hbm.at[pl.ds(sid * ROWS_PT, ROWS_PT)],
                    degsh.at[pl.ds(sid * ROWS_PT, ROWS_PT)])
    pltpu.sync_copy(ones_hbm.at[pl.ds(0, CHUNK)], onesv)
    pltpu.sync_copy(dst_hbm.at[wid], idxv)
    plsc.subcore_barrier()

    descs = [pltpu.async_copy(onesv, degsh.at[idxv.at[c]], sem, add=True)
             for c in range(CPW)]
    for d in descs:
        d.wait()
    plsc.subcore_barrier()
    pltpu.sync_copy(degsh.at[pl.ds(sid * ROWS_PT, ROWS_PT)],
                    out_hbm.at[cid, pl.ds(sid * ROWS_PT, ROWS_PT)])


@functools.cache
def _deg_call():
    mesh = plsc.VectorSubcoreMesh(core_axis_name="c", subcore_axis_name="s",
                                  num_cores=NC, num_subcores=NS)
    return pl.kernel(
        _deg_body, mesh=mesh,
        compiler_params=pltpu.CompilerParams(use_tc_tiling_on_sc=False),
        out_type=jax.ShapeDtypeStruct((NC, NPAD), jnp.float32),
        scratch_types=[
            pltpu.VMEM((CPW, CHUNK), jnp.int32),
            pltpu.VMEM((CHUNK,), jnp.float32),
            pltpu.VMEM_SHARED((NPAD,), jnp.float32),
            pltpu.SemaphoreType.DMA,
        ],
    )


NBUF = 8
LEAD = 4


def _edge_body(ht_hbm, src_hbm, dst_hbm, zeros_hbm, out_hbm,
               srcv, dstv, rows, accsh, gsem, ssem):
    cid = lax.axis_index("c")
    sid = lax.axis_index("s")
    wid = sid * NC + cid

    pltpu.sync_copy(zeros_hbm.at[pl.ds(sid * ROWS_PT, ROWS_PT)],
                    accsh.at[pl.ds(sid * ROWS_PT, ROWS_PT)])
    pltpu.sync_copy(src_hbm.at[wid], srcv)
    pltpu.sync_copy(dst_hbm.at[wid], dstv)
    plsc.subcore_barrier()

    gd = [None] * CPW
    sd = [None] * CPW
    for c in range(LEAD):
        gd[c] = pltpu.async_copy(ht_hbm.at[srcv.at[c]], rows[c % NBUF],
                                 gsem[c % NBUF])
    for c in range(CPW):
        cn = c + LEAD
        if cn < CPW:
            if cn - NBUF >= 0:
                sd[cn - NBUF].wait()
            gd[cn] = pltpu.async_copy(ht_hbm.at[srcv.at[cn]], rows[cn % NBUF],
                                      gsem[cn % NBUF])
        gd[c].wait()
        sd[c] = pltpu.async_copy(rows[c % NBUF], accsh.at[dstv.at[c]],
                                 ssem[c % NBUF], add=True)
    for c in range(max(0, CPW - NBUF), CPW):
        sd[c].wait()

    plsc.subcore_barrier()
    pltpu.sync_copy(accsh.at[pl.ds(sid * ROWS_PT, ROWS_PT)],
                    out_hbm.at[cid, pl.ds(sid * ROWS_PT, ROWS_PT)])


@functools.cache
def _edge_call():
    mesh = plsc.VectorSubcoreMesh(core_axis_name="c", subcore_axis_name="s",
                                  num_cores=NC, num_subcores=NS)
    return pl.kernel(
        _edge_body, mesh=mesh,
        compiler_params=pltpu.CompilerParams(use_tc_tiling_on_sc=False),
        out_type=jax.ShapeDtypeStruct((NC, NPAD, D), jnp.float32),
        scratch_types=[
            pltpu.VMEM((CPW, CHUNK), jnp.int32),
            pltpu.VMEM((CPW, CHUNK), jnp.int32),
            [pltpu.VMEM((CHUNK, D), jnp.float32) for _ in range(NBUF)],
            pltpu.VMEM_SHARED((NPAD, D), jnp.float32),
            [pltpu.SemaphoreType.DMA for _ in range(NBUF)],
            [pltpu.SemaphoreType.DMA for _ in range(NBUF)],
        ],
    )


def _prep_body(x_ref, w_ref, b_ref, d2_ref, ht_ref, dv_ref):
    deg = jnp.maximum(d2_ref[0] + d2_ref[1] + 1.0, 1.0)
    dinv = lax.rsqrt(deg)
    dv = jnp.broadcast_to(dinv[:, None], (NPAD, D))
    dv_ref[...] = dv
    hw = jnp.dot(x_ref[...], w_ref[...],
                 preferred_element_type=jnp.float32) + b_ref[...]
    ht_ref[...] = hw * dv


def _prep_call(xp, W0, b0, deg2):
    return pl.pallas_call(
        _prep_body,
        out_shape=[jax.ShapeDtypeStruct((NPAD, D), jnp.float32),
                   jax.ShapeDtypeStruct((NPAD, D), jnp.float32)],
    )(xp, W0, b0, deg2)


def _mid_body(acc_ref, htp_ref, dv_ref, m_ref, w_ref, b_ref, ht_ref, mo_ref):
    dv = dv_ref[...]
    h = jnp.maximum(dv * (acc_ref[0] + acc_ref[1] + htp_ref[...]), 0.0)
    mo_ref[...] = jnp.maximum(m_ref[...], h)
    ht_ref[...] = (jnp.dot(h, w_ref[...],
                           preferred_element_type=jnp.float32) + b_ref[...]) * dv


def _mid_call(acc, htp, dv, m, W, b):
    return pl.pallas_call(
        _mid_body,
        out_shape=[jax.ShapeDtypeStruct((NPAD, D), jnp.float32),
                   jax.ShapeDtypeStruct((NPAD, D), jnp.float32)],
    )(acc, htp, dv, m, W, b)


def _final_body(acc_ref, htp_ref, dv_ref, m_ref, w_ref, b_ref, out_ref):
    h = jnp.maximum(dv_ref[...] * (acc_ref[0] + acc_ref[1] + htp_ref[...]), 0.0)
    m = jnp.maximum(m_ref[...], h)
    o = jnp.dot(m, w_ref[...], preferred_element_type=jnp.float32) + b_ref[...]
    o = o - jnp.max(o, axis=1, keepdims=True)
    out_ref[...] = o - jnp.log(jnp.sum(jnp.exp(o), axis=1, keepdims=True))


def _final_call(acc, htp, dv, m, fcW, fcb):
    return pl.pallas_call(
        _final_body,
        out_shape=jax.ShapeDtypeStruct((NPAD, NCLS), jnp.float32),
    )(acc, htp, dv, m, fcW, fcb)


def kernel(x, edge_index, W0, b0, W1, b1, W2, b2, W3, b3, W4, b4, W5, b5,
           fcW, fcb):
    pad = (jnp.arange(EPAD - E, dtype=jnp.int32) % (NPAD - N)) + N
    srcp = jnp.concatenate([edge_index[0], pad]).reshape(NW, CPW, CHUNK)
    dstp = jnp.concatenate([edge_index[1], pad]).reshape(NW, CPW, CHUNK)
    xp = jnp.pad(x, ((0, NPAD - N), (0, 0)))
    zeros2 = jnp.zeros((NPAD, D), jnp.float32)
    zeros1 = jnp.zeros((NPAD,), jnp.float32)
    ones1 = jnp.ones((NPAD,), jnp.float32)

    deg2 = _deg_call()(dstp, ones1, zeros1)
    ht, dv = _prep_call(xp, W0, b0.reshape(1, D), deg2)
    m = jnp.zeros((NPAD, D), jnp.float32)
    for W, b in ((W1, b1), (W2, b2), (W3, b3), (W4, b4), (W5, b5)):
        acc = _edge_call()(ht, srcp, dstp, zeros2)
        ht, m = _mid_call(acc, ht, dv, m, W, b.reshape(1, D))
    acc = _edge_call()(ht, srcp, dstp, zeros2)
    out = _final_call(acc, ht, dv, m, fcW, fcb.reshape(1, NCLS))
    return out[:N]


# speedup vs baseline: 72.3460x; 1.5014x over previous
import functools

import jax
import jax.numpy as jnp
from jax import lax
from jax.experimental import pallas as pl
from jax.experimental.pallas import tpu as pltpu
from jax.experimental.pallas import tpu_sc as plsc

N = 10000
NPAD = 10240
NP8 = NPAD // 8
D = 16
NCLS = 64
NC = 2
NS = 16
NW = NC * NS
E = 320000
CHUNK = 128
CPW = (E + NW * CHUNK - 1) // (NW * CHUNK)
EPW = CPW * CHUNK
EPAD = EPW * NW
ROWS_PT = NPAD // NS
NBUF = 8
LEAD = 4


def _deg_body(dst_hbm, ones_hbm, zeros_hbm, out_hbm, idxv, onesv, degv, degx, degsh, sem):
    cid = lax.axis_index("c")
    sid = lax.axis_index("s")
    wid = sid * NC + cid

    pltpu.sync_copy(zeros_hbm.at[pl.ds(sid * ROWS_PT, ROWS_PT)],
                    degsh.at[pl.ds(sid * ROWS_PT, ROWS_PT)])
    pltpu.sync_copy(ones_hbm.at[pl.ds(0, CHUNK)], onesv)
    pltpu.sync_copy(dst_hbm.at[wid], idxv)
    plsc.subcore_barrier()

    descs = [pltpu.async_copy(onesv, degsh.at[idxv.at[c]], sem, add=True)
             for c in range(CPW)]
    for d_ in descs:
        d_.wait()
    plsc.subcore_barrier()

    pltpu.sync_copy(degsh.at[pl.ds(sid * ROWS_PT, ROWS_PT)], degv)

    def expand(n, carry):
        idx = jnp.full((D,), n, jnp.int32)
        degx[n] = plsc.load_gather(degv, [idx])
        return carry

    lax.fori_loop(0, ROWS_PT, expand, 0)
    pltpu.sync_copy(degx, out_hbm.at[cid, pl.ds(sid * ROWS_PT, ROWS_PT)])


@functools.cache
def _deg_call():
    mesh = plsc.VectorSubcoreMesh(core_axis_name="c", subcore_axis_name="s",
                                  num_cores=NC, num_subcores=NS)
    return pl.kernel(
        _deg_body, mesh=mesh,
        compiler_params=pltpu.CompilerParams(use_tc_tiling_on_sc=False,
                                             needs_layout_passes=False),
        out_type=jax.ShapeDtypeStruct((NC, NPAD, D), jnp.float32),
        scratch_types=[
            pltpu.VMEM((CPW, CHUNK), jnp.int32),
            pltpu.VMEM((CHUNK,), jnp.float32),
            pltpu.VMEM((ROWS_PT,), jnp.float32),
            pltpu.VMEM((ROWS_PT, D), jnp.float32),
            pltpu.VMEM_SHARED((NPAD,), jnp.float32),
            pltpu.SemaphoreType.DMA,
        ],
    )


def _edge_body(ht_hbm, src_hbm, dst_hbm, zeros_hbm, out_hbm,
               srcv, dstv, rows, accsh, gsem, ssem):
    cid = lax.axis_index("c")
    sid = lax.axis_index("s")
    wid = sid * NC + cid

    pltpu.sync_copy(zeros_hbm.at[pl.ds(sid * ROWS_PT, ROWS_PT)],
                    accsh.at[pl.ds(sid * ROWS_PT, ROWS_PT)])
    pltpu.sync_copy(src_hbm.at[wid], srcv)
    pltpu.sync_copy(dst_hbm.at[wid], dstv)
    plsc.subcore_barrier()

    gd = [None] * CPW
    sd = [None] * CPW
    for c in range(LEAD):
        gd[c] = pltpu.async_copy(ht_hbm.at[srcv.at[c]], rows[c % NBUF],
                                 gsem[c % NBUF])
    for c in range(CPW):
        cn = c + LEAD
        if cn < CPW:
            if cn - NBUF >= 0:
                sd[cn - NBUF].wait()
            gd[cn] = pltpu.async_copy(ht_hbm.at[srcv.at[cn]], rows[cn % NBUF],
                                      gsem[cn % NBUF])
        gd[c].wait()
        sd[c] = pltpu.async_copy(rows[c % NBUF], accsh.at[dstv.at[c]],
                                 ssem[c % NBUF], add=True)
    for c in range(max(0, CPW - NBUF), CPW):
        sd[c].wait()

    plsc.subcore_barrier()
    pltpu.sync_copy(accsh.at[pl.ds(sid * ROWS_PT, ROWS_PT)],
                    out_hbm.at[cid, pl.ds(sid * ROWS_PT, ROWS_PT)])


@functools.cache
def _edge_call():
    mesh = plsc.VectorSubcoreMesh(core_axis_name="c", subcore_axis_name="s",
                                  num_cores=NC, num_subcores=NS)
    return pl.kernel(
        _edge_body, mesh=mesh,
        compiler_params=pltpu.CompilerParams(use_tc_tiling_on_sc=False,
                                             needs_layout_passes=False),
        out_type=jax.ShapeDtypeStruct((NC, NPAD, D), jnp.float32),
        scratch_types=[
            pltpu.VMEM((CPW, CHUNK), jnp.int32),
            pltpu.VMEM((CPW, CHUNK), jnp.int32),
            [pltpu.VMEM((CHUNK, D), jnp.float32) for _ in range(NBUF)],
            pltpu.VMEM_SHARED((NPAD, D), jnp.float32),
            [pltpu.SemaphoreType.DMA for _ in range(NBUF)],
            [pltpu.SemaphoreType.DMA for _ in range(NBUF)],
        ],
    )


def _prep_body(x_ref, w_ref, b_ref, d2_ref, ht_ref, dv_ref):
    dv = lax.rsqrt(jnp.maximum(d2_ref[0] + d2_ref[1] + 1.0, 1.0))
    dv_ref[...] = dv
    hw = jnp.dot(x_ref[...], w_ref[...],
                 preferred_element_type=jnp.float32) + b_ref[...]
    ht_ref[...] = hw * dv


def _prep_call(xp, W0t, b0t, deg2):
    return pl.pallas_call(
        _prep_body,
        out_shape=[jax.ShapeDtypeStruct((NP8, 128), jnp.float32),
                   jax.ShapeDtypeStruct((NP8, 128), jnp.float32)],
    )(xp, W0t, b0t, deg2)


def _mid_body(acc_ref, htp_ref, dv_ref, m_ref, w_ref, b_ref, ht_ref, mo_ref):
    dv = dv_ref[...]
    h = jnp.maximum(dv * (acc_ref[0] + acc_ref[1] + htp_ref[...]), 0.0)
    mo_ref[...] = jnp.maximum(m_ref[...], h)
    ht_ref[...] = (jnp.dot(h, w_ref[...],
                           preferred_element_type=jnp.float32) + b_ref[...]) * dv


def _mid_call(acc, htp, dv, m, Wt, bt):
    return pl.pallas_call(
        _mid_body,
        out_shape=[jax.ShapeDtypeStruct((NP8, 128), jnp.float32),
                   jax.ShapeDtypeStruct((NP8, 128), jnp.float32)],
    )(acc, htp, dv, m, Wt, bt)


def _final_body(acc_ref, htp_ref, dv_ref, m_ref, w_ref, b_ref, red_ref, bc_ref,
                out_ref):
    h = jnp.maximum(dv_ref[...] * (acc_ref[0] + acc_ref[1] + htp_ref[...]), 0.0)
    m = jnp.maximum(m_ref[...], h)
    o = jnp.dot(m, w_ref[...], preferred_element_type=jnp.float32) + b_ref[...]
    o = o - jnp.max(o)
    eo = jnp.exp(o)
    s = jnp.dot(eo, red_ref[...], preferred_element_type=jnp.float32)
    lse = jnp.dot(jnp.log(s), bc_ref[...],
                  preferred_element_type=jnp.float32)
    out_ref[...] = o - lse


def _final_call(acc, htp, dv, m, fcWt, fcbt, red, bc):
    return pl.pallas_call(
        _final_body,
        out_shape=jax.ShapeDtypeStruct((NP8, 8 * NCLS), jnp.float32),
    )(acc, htp, dv, m, fcWt, fcbt, red, bc)


def kernel(x, edge_index, W0, b0, W1, b1, W2, b2, W3, b3, W4, b4, W5, b5,
           fcW, fcb):
    f32 = jnp.float32
    eye8 = jnp.eye(8, dtype=f32)
    pad = (jnp.arange(EPAD - E, dtype=jnp.int32) % (NPAD - N)) + N
    srcp = jnp.concatenate([edge_index[0], pad]).reshape(NW, CPW, CHUNK)
    dstp = jnp.concatenate([edge_index[1], pad]).reshape(NW, CPW, CHUNK)
    xp = jnp.pad(x, ((0, NPAD - N), (0, 0))).reshape(NP8, 8 * 128)
    zeros2 = jnp.zeros((NPAD, D), f32)
    zeros1 = jnp.zeros((NPAD,), f32)
    ones1 = jnp.ones((NPAD,), f32)
    W0t = jnp.kron(eye8, W0)
    b0t = jnp.tile(b0, 8).reshape(1, 128)
    fcWt = jnp.kron(eye8, fcW)
    fcbt = jnp.tile(fcb, 8).reshape(1, 8 * NCLS)
    red = jnp.kron(eye8, jnp.ones((NCLS, 1), f32))
    bc = jnp.kron(eye8, jnp.ones((1, NCLS), f32))

    deg2 = _deg_call()(dstp, ones1, zeros1).reshape(NC, NP8, 128)
    ht, dv = _prep_call(xp, W0t, b0t, deg2)
    m = jnp.zeros((NP8, 128), f32)
    for W, b in ((W1, b1), (W2, b2), (W3, b3), (W4, b4), (W5, b5)):
        acc = _edge_call()(ht.reshape(NPAD, D), srcp, dstp, zeros2)
        ht, m = _mid_call(acc.reshape(NC, NP8, 128), ht, dv, m,
                          jnp.kron(eye8, W), jnp.tile(b, 8).reshape(1, 128))
    acc = _edge_call()(ht.reshape(NPAD, D), srcp, dstp, zeros2)
    out = _final_call(acc.reshape(NC, NP8, 128), ht, dv, m, fcWt, fcbt, red, bc)
    return out.reshape(NPAD, NCLS)[:N]


# speedup vs baseline: 79.4176x; 1.0977x over previous
import functools

import jax
import jax.numpy as jnp
from jax import lax
from jax.experimental import pallas as pl
from jax.experimental.pallas import tpu as pltpu
from jax.experimental.pallas import tpu_sc as plsc

N = 10000
NPAD = 10240
NP8 = NPAD // 8
D = 16
NCLS = 64
NC = 2
NS = 16
NW = NC * NS
E = 320000
CHUNK = 128
CPW = (E + NW * CHUNK - 1) // (NW * CHUNK)
EPW = CPW * CHUNK
EPAD = EPW * NW
ROWS_PT = NPAD // NS
NBUF = 8
LEAD = 4


def _deg_body(dst_hbm, ones_hbm, zeros_hbm, out_hbm, idxv, onesv, degv, degx, degsh, sem):
    cid = lax.axis_index("c")
    sid = lax.axis_index("s")
    wid = sid * NC + cid

    pltpu.sync_copy(zeros_hbm.at[pl.ds(sid * ROWS_PT, ROWS_PT)],
                    degsh.at[pl.ds(sid * ROWS_PT, ROWS_PT)])
    pltpu.sync_copy(ones_hbm.at[pl.ds(0, CHUNK)], onesv)
    pltpu.sync_copy(dst_hbm.at[wid], idxv)
    plsc.subcore_barrier()

    descs = [pltpu.async_copy(onesv, degsh.at[idxv.at[c]], sem, add=True)
             for c in range(CPW)]
    for d_ in descs:
        d_.wait()
    plsc.subcore_barrier()

    pltpu.sync_copy(degsh.at[pl.ds(sid * ROWS_PT, ROWS_PT)], degv)

    def expand(n, carry):
        idx = jnp.full((D,), n, jnp.int32)
        degx[n] = plsc.load_gather(degv, [idx])
        return carry

    lax.fori_loop(0, ROWS_PT, expand, 0)
    pltpu.sync_copy(degx, out_hbm.at[cid, pl.ds(sid * ROWS_PT, ROWS_PT)])


@functools.cache
def _deg_call():
    mesh = plsc.VectorSubcoreMesh(core_axis_name="c", subcore_axis_name="s",
                                  num_cores=NC, num_subcores=NS)
    return pl.kernel(
        _deg_body, mesh=mesh,
        compiler_params=pltpu.CompilerParams(use_tc_tiling_on_sc=False,
                                             needs_layout_passes=False),
        out_type=jax.ShapeDtypeStruct((NC, NPAD, D), jnp.float32),
        scratch_types=[
            pltpu.VMEM((CPW, CHUNK), jnp.int32),
            pltpu.VMEM((CHUNK,), jnp.float32),
            pltpu.VMEM((ROWS_PT,), jnp.float32),
            pltpu.VMEM((ROWS_PT, D), jnp.float32),
            pltpu.VMEM_SHARED((NPAD,), jnp.float32),
            pltpu.SemaphoreType.DMA,
        ],
    )


def _edge_body(ht_hbm, src_hbm, dst_hbm, zeros_hbm, out_hbm,
               srcv, dstv, rows, accsh, htsh, gsem, ssem):
    cid = lax.axis_index("c")
    sid = lax.axis_index("s")
    wid = sid * NC + cid

    pltpu.sync_copy(zeros_hbm.at[pl.ds(sid * ROWS_PT, ROWS_PT)],
                    accsh.at[pl.ds(sid * ROWS_PT, ROWS_PT)])
    pltpu.sync_copy(ht_hbm.at[pl.ds(sid * ROWS_PT, ROWS_PT)],
                    htsh.at[pl.ds(sid * ROWS_PT, ROWS_PT)])
    pltpu.sync_copy(src_hbm.at[wid], srcv)
    pltpu.sync_copy(dst_hbm.at[wid], dstv)
    plsc.subcore_barrier()

    gd = [None] * CPW
    sd = [None] * CPW
    for c in range(LEAD):
        gd[c] = pltpu.async_copy(htsh.at[srcv.at[c]], rows[c % NBUF],
                                 gsem[c % NBUF])
    for c in range(CPW):
        cn = c + LEAD
        if cn < CPW:
            if cn - NBUF >= 0:
                sd[cn - NBUF].wait()
            gd[cn] = pltpu.async_copy(htsh.at[srcv.at[cn]], rows[cn % NBUF],
                                      gsem[cn % NBUF])
        gd[c].wait()
        sd[c] = pltpu.async_copy(rows[c % NBUF], accsh.at[dstv.at[c]],
                                 ssem[c % NBUF], add=True)
    for c in range(max(0, CPW - NBUF), CPW):
        sd[c].wait()

    plsc.subcore_barrier()
    pltpu.sync_copy(accsh.at[pl.ds(sid * ROWS_PT, ROWS_PT)],
                    out_hbm.at[cid, pl.ds(sid * ROWS_PT, ROWS_PT)])


@functools.cache
def _edge_call():
    mesh = plsc.VectorSubcoreMesh(core_axis_name="c", subcore_axis_name="s",
                                  num_cores=NC, num_subcores=NS)
    return pl.kernel(
        _edge_body, mesh=mesh,
        compiler_params=pltpu.CompilerParams(use_tc_tiling_on_sc=False,
                                             needs_layout_passes=False),
        out_type=jax.ShapeDtypeStruct((NC, NPAD, D), jnp.float32),
        scratch_types=[
            pltpu.VMEM((CPW, CHUNK), jnp.int32),
            pltpu.VMEM((CPW, CHUNK), jnp.int32),
            [pltpu.VMEM((CHUNK, D), jnp.float32) for _ in range(NBUF)],
            pltpu.VMEM_SHARED((NPAD, D), jnp.float32),
            pltpu.VMEM_SHARED((NPAD, D), jnp.float32),
            [pltpu.SemaphoreType.DMA for _ in range(NBUF)],
            [pltpu.SemaphoreType.DMA for _ in range(NBUF)],
        ],
    )


def _prep_body(x_ref, w_ref, b_ref, d2_ref, ht_ref, dv_ref):
    dv = lax.rsqrt(jnp.maximum(d2_ref[0] + d2_ref[1] + 1.0, 1.0))
    dv_ref[...] = dv
    hw = jnp.dot(x_ref[...], w_ref[...],
                 preferred_element_type=jnp.float32) + b_ref[...]
    ht_ref[...] = hw * dv


def _prep_call(xp, W0t, b0t, deg2):
    return pl.pallas_call(
        _prep_body,
        out_shape=[jax.ShapeDtypeStruct((NP8, 128), jnp.float32),
                   jax.ShapeDtypeStruct((NP8, 128), jnp.float32)],
    )(xp, W0t, b0t, deg2)


def _mid_body(acc_ref, htp_ref, dv_ref, m_ref, w_ref, b_ref, ht_ref, mo_ref):
    dv = dv_ref[...]
    h = jnp.maximum(dv * (acc_ref[0] + acc_ref[1] + htp_ref[...]), 0.0)
    mo_ref[...] = jnp.maximum(m_ref[...], h)
    ht_ref[...] = (jnp.dot(h, w_ref[...],
                           preferred_element_type=jnp.float32) + b_ref[...]) * dv


def _mid_call(acc, htp, dv, m, Wt, bt):
    return pl.pallas_call(
        _mid_body,
        out_shape=[jax.ShapeDtypeStruct((NP8, 128), jnp.float32),
                   jax.ShapeDtypeStruct((NP8, 128), jnp.float32)],
    )(acc, htp, dv, m, Wt, bt)


def _final_body(acc_ref, htp_ref, dv_ref, m_ref, w_ref, b_ref, red_ref, bc_ref,
                out_ref):
    h = jnp.maximum(dv_ref[...] * (acc_ref[0] + acc_ref[1] + htp_ref[...]), 0.0)
    m = jnp.maximum(m_ref[...], h)
    o = jnp.dot(m, w_ref[...], preferred_element_type=jnp.float32) + b_ref[...]
    o = o - jnp.max(o)
    eo = jnp.exp(o)
    s = jnp.dot(eo, red_ref[...], preferred_element_type=jnp.float32)
    lse = jnp.dot(jnp.log(s), bc_ref[...],
                  preferred_element_type=jnp.float32)
    out_ref[...] = o - lse


def _final_call(acc, htp, dv, m, fcWt, fcbt, red, bc):
    return pl.pallas_call(
        _final_body,
        out_shape=jax.ShapeDtypeStruct((NP8, 8 * NCLS), jnp.float32),
    )(acc, htp, dv, m, fcWt, fcbt, red, bc)


def kernel(x, edge_index, W0, b0, W1, b1, W2, b2, W3, b3, W4, b4, W5, b5,
           fcW, fcb):
    f32 = jnp.float32
    eye8 = jnp.eye(8, dtype=f32)
    pad = (jnp.arange(EPAD - E, dtype=jnp.int32) % (NPAD - N)) + N
    srcp = jnp.concatenate([edge_index[0], pad]).reshape(NW, CPW, CHUNK)
    dstp = jnp.concatenate([edge_index[1], pad]).reshape(NW, CPW, CHUNK)
    xp = jnp.pad(x, ((0, NPAD - N), (0, 0))).reshape(NP8, 8 * 128)
    zeros2 = jnp.zeros((NPAD, D), f32)
    zeros1 = jnp.zeros((NPAD,), f32)
    ones1 = jnp.ones((NPAD,), f32)
    W0t = jnp.kron(eye8, W0)
    b0t = jnp.tile(b0, 8).reshape(1, 128)
    fcWt = jnp.kron(eye8, fcW)
    fcbt = jnp.tile(fcb, 8).reshape(1, 8 * NCLS)
    red = jnp.kron(eye8, jnp.ones((NCLS, 1), f32))
    bc = jnp.kron(eye8, jnp.ones((1, NCLS), f32))

    deg2 = _deg_call()(dstp, ones1, zeros1).reshape(NC, NP8, 128)
    ht, dv = _prep_call(xp, W0t, b0t, deg2)
    m = jnp.zeros((NP8, 128), f32)
    for W, b in ((W1, b1), (W2, b2), (W3, b3), (W4, b4), (W5, b5)):
        acc = _edge_call()(ht.reshape(NPAD, D), srcp, dstp, zeros2)
        ht, m = _mid_call(acc.reshape(NC, NP8, 128), ht, dv, m,
                          jnp.kron(eye8, W), jnp.tile(b, 8).reshape(1, 128))
    acc = _edge_call()(ht.reshape(NPAD, D), srcp, dstp, zeros2)
    out = _final_call(acc.reshape(NC, NP8, 128), ht, dv, m, fcWt, fcbt, red, bc)
    return out.reshape(NPAD, NCLS)[:N]


# speedup vs baseline: 84.4524x; 1.0634x over previous
import functools

import jax
import jax.numpy as jnp
from jax import lax
from jax.experimental import pallas as pl
from jax.experimental.pallas import tpu as pltpu
from jax.experimental.pallas import tpu_sc as plsc

N = 10000
NPAD = 10240
NP8 = NPAD // 8
D = 16
NCLS = 64
NC = 2
NS = 16
NW = NC * NS
E = 320000
CHUNK = 128
CPW = (E + NW * CHUNK - 1) // (NW * CHUNK)
EPW = CPW * CHUNK
EPAD = EPW * NW
ROWS_PT = NPAD // NS
NBUF = 12
LEAD = 6


def _deg_body(dst_hbm, ones_hbm, zeros_hbm, out_hbm, idxv, onesv, degv, degx, degsh, sem):
    cid = lax.axis_index("c")
    sid = lax.axis_index("s")
    wid = sid * NC + cid

    pltpu.sync_copy(zeros_hbm.at[pl.ds(sid * ROWS_PT, ROWS_PT)],
                    degsh.at[pl.ds(sid * ROWS_PT, ROWS_PT)])
    pltpu.sync_copy(ones_hbm.at[pl.ds(0, CHUNK)], onesv)
    pltpu.sync_copy(dst_hbm.at[wid], idxv)
    plsc.subcore_barrier()

    descs = [pltpu.async_copy(onesv, degsh.at[idxv.at[c]], sem, add=True)
             for c in range(CPW)]
    for d_ in descs:
        d_.wait()
    plsc.subcore_barrier()

    pltpu.sync_copy(degsh.at[pl.ds(sid * ROWS_PT, ROWS_PT)], degv)

    def expand(n, carry):
        idx = jnp.full((D,), n, jnp.int32)
        degx[n] = plsc.load_gather(degv, [idx])
        return carry

    lax.fori_loop(0, ROWS_PT, expand, 0)
    pltpu.sync_copy(degx, out_hbm.at[cid, pl.ds(sid * ROWS_PT, ROWS_PT)])


@functools.cache
def _deg_call():
    mesh = plsc.VectorSubcoreMesh(core_axis_name="c", subcore_axis_name="s",
                                  num_cores=NC, num_subcores=NS)
    return pl.kernel(
        _deg_body, mesh=mesh,
        compiler_params=pltpu.CompilerParams(use_tc_tiling_on_sc=False,
                                             needs_layout_passes=False),
        out_type=jax.ShapeDtypeStruct((NC, NPAD, D), jnp.float32),
        scratch_types=[
            pltpu.VMEM((CPW, CHUNK), jnp.int32),
            pltpu.VMEM((CHUNK,), jnp.float32),
            pltpu.VMEM((ROWS_PT,), jnp.float32),
            pltpu.VMEM((ROWS_PT, D), jnp.float32),
            pltpu.VMEM_SHARED((NPAD,), jnp.float32),
            pltpu.SemaphoreType.DMA,
        ],
    )


def _edge_body(ht_hbm, src_hbm, dst_hbm, out_hbm,
               srcv, dstv, rows, accsh, htsh, gsem, ssem):
    cid = lax.axis_index("c")
    sid = lax.axis_index("s")
    wid = sid * NC + cid

    def zrow(r, carry):
        rows[0][r] = jnp.zeros((D,), jnp.float32)
        return carry

    lax.fori_loop(0, CHUNK, zrow, 0)
    for z in range(ROWS_PT // CHUNK):
        pltpu.sync_copy(rows[0],
                        accsh.at[pl.ds(sid * ROWS_PT + z * CHUNK, CHUNK)])
    pltpu.sync_copy(ht_hbm.at[pl.ds(sid * ROWS_PT, ROWS_PT)],
                    htsh.at[pl.ds(sid * ROWS_PT, ROWS_PT)])
    pltpu.sync_copy(src_hbm.at[wid], srcv)
    pltpu.sync_copy(dst_hbm.at[wid], dstv)
    plsc.subcore_barrier()

    gd = [None] * CPW
    sd = [None] * CPW
    for c in range(LEAD):
        gd[c] = pltpu.async_copy(htsh.at[srcv.at[c]], rows[c % NBUF],
                                 gsem[c % NBUF])
    for c in range(CPW):
        cn = c + LEAD
        if cn < CPW:
            if cn - NBUF >= 0:
                sd[cn - NBUF].wait()
            gd[cn] = pltpu.async_copy(htsh.at[srcv.at[cn]], rows[cn % NBUF],
                                      gsem[cn % NBUF])
        gd[c].wait()
        sd[c] = pltpu.async_copy(rows[c % NBUF], accsh.at[dstv.at[c]],
                                 ssem[c % NBUF], add=True)
    for c in range(max(0, CPW - NBUF), CPW):
        sd[c].wait()

    plsc.subcore_barrier()
    pltpu.sync_copy(accsh.at[pl.ds(sid * ROWS_PT, ROWS_PT)],
                    out_hbm.at[cid, pl.ds(sid * ROWS_PT, ROWS_PT)])


@functools.cache
def _edge_call():
    mesh = plsc.VectorSubcoreMesh(core_axis_name="c", subcore_axis_name="s",
                                  num_cores=NC, num_subcores=NS)
    return pl.kernel(
        _edge_body, mesh=mesh,
        compiler_params=pltpu.CompilerParams(use_tc_tiling_on_sc=False,
                                             needs_layout_passes=False),
        out_type=jax.ShapeDtypeStruct((NC, NPAD, D), jnp.float32),
        scratch_types=[
            pltpu.VMEM((CPW, CHUNK), jnp.int32),
            pltpu.VMEM((CPW, CHUNK), jnp.int32),
            [pltpu.VMEM((CHUNK, D), jnp.float32) for _ in range(NBUF)],
            pltpu.VMEM_SHARED((NPAD, D), jnp.float32),
            pltpu.VMEM_SHARED((NPAD, D), jnp.float32),
            [pltpu.SemaphoreType.DMA for _ in range(NBUF)],
            [pltpu.SemaphoreType.DMA for _ in range(NBUF)],
        ],
    )


def _prep_body(x_ref, w_ref, b_ref, d2_ref, ht_ref, dv_ref):
    dv = lax.rsqrt(jnp.maximum(d2_ref[0] + d2_ref[1] + 1.0, 1.0))
    dv_ref[...] = dv
    hw = jnp.dot(x_ref[...], w_ref[...],
                 preferred_element_type=jnp.float32) + b_ref[...]
    ht_ref[...] = hw * dv


def _prep_call(xp, W0t, b0t, deg2):
    return pl.pallas_call(
        _prep_body,
        out_shape=[jax.ShapeDtypeStruct((NP8, 128), jnp.float32),
                   jax.ShapeDtypeStruct((NP8, 128), jnp.float32)],
    )(xp, W0t, b0t, deg2)


def _mid_body(acc_ref, htp_ref, dv_ref, m_ref, w_ref, b_ref, ht_ref, mo_ref):
    dv = dv_ref[...]
    h = jnp.maximum(dv * (acc_ref[0] + acc_ref[1] + htp_ref[...]), 0.0)
    mo_ref[...] = jnp.maximum(m_ref[...], h)
    ht_ref[...] = (jnp.dot(h, w_ref[...],
                           preferred_element_type=jnp.float32) + b_ref[...]) * dv


def _mid_call(acc, htp, dv, m, Wt, bt):
    return pl.pallas_call(
        _mid_body,
        out_shape=[jax.ShapeDtypeStruct((NP8, 128), jnp.float32),
                   jax.ShapeDtypeStruct((NP8, 128), jnp.float32)],
    )(acc, htp, dv, m, Wt, bt)


def _final_body(acc_ref, htp_ref, dv_ref, m_ref, w_ref, b_ref, red_ref, bc_ref,
                out_ref):
    h = jnp.maximum(dv_ref[...] * (acc_ref[0] + acc_ref[1] + htp_ref[...]), 0.0)
    m = jnp.maximum(m_ref[...], h)
    o = jnp.dot(m, w_ref[...], preferred_element_type=jnp.float32) + b_ref[...]
    o = o - jnp.max(o)
    eo = jnp.exp(o)
    s = jnp.dot(eo, red_ref[...], preferred_element_type=jnp.float32)
    lse = jnp.dot(jnp.log(s), bc_ref[...],
                  preferred_element_type=jnp.float32)
    out_ref[...] = o - lse


def _final_call(acc, htp, dv, m, fcWt, fcbt, red, bc):
    return pl.pallas_call(
        _final_body,
        out_shape=jax.ShapeDtypeStruct((NP8, 8 * NCLS), jnp.float32),
    )(acc, htp, dv, m, fcWt, fcbt, red, bc)


def kernel(x, edge_index, W0, b0, W1, b1, W2, b2, W3, b3, W4, b4, W5, b5,
           fcW, fcb):
    f32 = jnp.float32
    eye8 = jnp.eye(8, dtype=f32)
    pad = (jnp.arange(EPAD - E, dtype=jnp.int32) % (NPAD - N)) + N
    srcp = jnp.concatenate([edge_index[0], pad]).reshape(NW, CPW, CHUNK)
    dstp = jnp.concatenate([edge_index[1], pad]).reshape(NW, CPW, CHUNK)
    xp = jnp.pad(x, ((0, NPAD - N), (0, 0))).reshape(NP8, 8 * 128)
    zeros1 = jnp.zeros((NPAD,), f32)
    ones1 = jnp.ones((NPAD,), f32)
    W0t = jnp.kron(eye8, W0)
    b0t = jnp.tile(b0, 8).reshape(1, 128)
    fcWt = jnp.kron(eye8, fcW)
    fcbt = jnp.tile(fcb, 8).reshape(1, 8 * NCLS)
    red = jnp.kron(eye8, jnp.ones((NCLS, 1), f32))
    bc = jnp.kron(eye8, jnp.ones((1, NCLS), f32))

    deg2 = _deg_call()(dstp, ones1, zeros1).reshape(NC, NP8, 128)
    ht, dv = _prep_call(xp, W0t, b0t, deg2)
    m = jnp.zeros((NP8, 128), f32)
    Wts = [(jnp.kron(eye8, W), jnp.tile(b, 8).reshape(1, 128))
           for W, b in ((W1, b1), (W2, b2), (W3, b3), (W4, b4), (W5, b5))]
    for Wt, bt in Wts:
        acc = _edge_call()(ht.reshape(NPAD, D), srcp, dstp)
        ht, m = _mid_call(acc.reshape(NC, NP8, 128), ht, dv, m, Wt, bt)
    acc = _edge_call()(ht.reshape(NPAD, D), srcp, dstp)
    out = _final_call(acc.reshape(NC, NP8, 128), ht, dv, m, fcWt, fcbt, red, bc)
    return out.reshape(NPAD, NCLS)[:N]


# speedup vs baseline: 90.5282x; 1.0719x over previous
import functools

import jax
import jax.numpy as jnp
from jax import lax
from jax.experimental import pallas as pl
from jax.experimental.pallas import tpu as pltpu
from jax.experimental.pallas import tpu_sc as plsc

N = 10000
NPAD = 10240
NP8 = NPAD // 8
D = 16
NCLS = 64
NC = 2
NS = 16
NW = NC * NS
E = 320000
CHUNK = 128
CPW = (E + NW * CHUNK - 1) // (NW * CHUNK)
EPW = CPW * CHUNK
EPAD = EPW * NW
ROWS_PT = NPAD // NS
NBUF = 12
LEAD = 6


def _deg_body(dst_hbm, ones_hbm, zeros_hbm, out_hbm, idxv, onesv, degv, degx, degsh, sem):
    cid = lax.axis_index("c")
    sid = lax.axis_index("s")
    wid = sid * NC + cid

    stage = [pltpu.async_copy(zeros_hbm.at[pl.ds(sid * ROWS_PT, ROWS_PT)],
                              degsh.at[pl.ds(sid * ROWS_PT, ROWS_PT)], sem),
             pltpu.async_copy(ones_hbm.at[pl.ds(0, CHUNK)], onesv, sem),
             pltpu.async_copy(dst_hbm.at[wid], idxv, sem)]
    for d_ in stage:
        d_.wait()
    plsc.subcore_barrier()

    descs = [pltpu.async_copy(onesv, degsh.at[idxv.at[c]], sem, add=True)
             for c in range(CPW)]
    for d_ in descs:
        d_.wait()
    plsc.subcore_barrier()

    pltpu.sync_copy(degsh.at[pl.ds(sid * ROWS_PT, ROWS_PT)], degv)

    def expand(n, carry):
        idx = jnp.full((D,), n, jnp.int32)
        degx[n] = plsc.load_gather(degv, [idx])
        return carry

    lax.fori_loop(0, ROWS_PT, expand, 0)
    pltpu.sync_copy(degx, out_hbm.at[cid, pl.ds(sid * ROWS_PT, ROWS_PT)])


@functools.cache
def _deg_call():
    mesh = plsc.VectorSubcoreMesh(core_axis_name="c", subcore_axis_name="s",
                                  num_cores=NC, num_subcores=NS)
    return pl.kernel(
        _deg_body, mesh=mesh,
        compiler_params=pltpu.CompilerParams(use_tc_tiling_on_sc=False,
                                             needs_layout_passes=False),
        out_type=jax.ShapeDtypeStruct((NC, NPAD, D), jnp.float32),
        scratch_types=[
            pltpu.VMEM((CPW, CHUNK), jnp.int32),
            pltpu.VMEM((CHUNK,), jnp.float32),
            pltpu.VMEM((ROWS_PT,), jnp.float32),
            pltpu.VMEM((ROWS_PT, D), jnp.float32),
            pltpu.VMEM_SHARED((NPAD,), jnp.float32),
            pltpu.SemaphoreType.DMA,
        ],
    )


def _edge_body(ht_hbm, src_hbm, dst_hbm, out_hbm,
               srcv, dstv, rows, accsh, htsh, gsem, ssem):
    cid = lax.axis_index("c")
    sid = lax.axis_index("s")
    wid = sid * NC + cid

    def zrow(r, carry):
        rows[0][r] = jnp.zeros((D,), jnp.float32)
        return carry

    lax.fori_loop(0, CHUNK, zrow, 0)
    stage = [pltpu.async_copy(rows[0],
                              accsh.at[pl.ds(sid * ROWS_PT + z * CHUNK, CHUNK)],
                              gsem[0])
             for z in range(ROWS_PT // CHUNK)]
    stage.append(pltpu.async_copy(ht_hbm.at[pl.ds(sid * ROWS_PT, ROWS_PT)],
                                  htsh.at[pl.ds(sid * ROWS_PT, ROWS_PT)],
                                  gsem[1]))
    stage.append(pltpu.async_copy(src_hbm.at[wid], srcv, gsem[2]))
    stage.append(pltpu.async_copy(dst_hbm.at[wid], dstv, gsem[3]))
    for d_ in stage:
        d_.wait()
    plsc.subcore_barrier()

    gd = [None] * CPW
    sd = [None] * CPW
    for c in range(LEAD):
        gd[c] = pltpu.async_copy(htsh.at[srcv.at[c]], rows[c % NBUF],
                                 gsem[c % NBUF])
    for c in range(CPW):
        cn = c + LEAD
        if cn < CPW:
            if cn - NBUF >= 0:
                sd[cn - NBUF].wait()
            gd[cn] = pltpu.async_copy(htsh.at[srcv.at[cn]], rows[cn % NBUF],
                                      gsem[cn % NBUF])
        gd[c].wait()
        sd[c] = pltpu.async_copy(rows[c % NBUF], accsh.at[dstv.at[c]],
                                 ssem[c % NBUF], add=True)
    for c in range(max(0, CPW - NBUF), CPW):
        sd[c].wait()

    plsc.subcore_barrier()
    pltpu.sync_copy(accsh.at[pl.ds(sid * ROWS_PT, ROWS_PT)],
                    out_hbm.at[cid, pl.ds(sid * ROWS_PT, ROWS_PT)])


@functools.cache
def _edge_call():
    mesh = plsc.VectorSubcoreMesh(core_axis_name="c", subcore_axis_name="s",
                                  num_cores=NC, num_subcores=NS)
    return pl.kernel(
        _edge_body, mesh=mesh,
        compiler_params=pltpu.CompilerParams(use_tc_tiling_on_sc=False,
                                             needs_layout_passes=False),
        out_type=jax.ShapeDtypeStruct((NC, NPAD, D), jnp.float32),
        scratch_types=[
            pltpu.VMEM((CPW, CHUNK), jnp.int32),
            pltpu.VMEM((CPW, CHUNK), jnp.int32),
            [pltpu.VMEM((CHUNK, D), jnp.float32) for _ in range(NBUF)],
            pltpu.VMEM_SHARED((NPAD, D), jnp.float32),
            pltpu.VMEM_SHARED((NPAD, D), jnp.float32),
            [pltpu.SemaphoreType.DMA for _ in range(NBUF)],
            [pltpu.SemaphoreType.DMA for _ in range(NBUF)],
        ],
    )


def _prep_body(x_ref, w_ref, b_ref, d2_ref, ht_ref, dv_ref):
    dv = lax.rsqrt(jnp.maximum(d2_ref[0] + d2_ref[1] + 1.0, 1.0))
    dv_ref[...] = dv
    hw = jnp.dot(x_ref[...], w_ref[...],
                 preferred_element_type=jnp.float32) + b_ref[...]
    ht_ref[...] = hw * dv


def _prep_call(xp, W0t, b0t, deg2):
    return pl.pallas_call(
        _prep_body,
        out_shape=[jax.ShapeDtypeStruct((NP8, 128), jnp.float32),
                   jax.ShapeDtypeStruct((NP8, 128), jnp.float32)],
    )(xp, W0t, b0t, deg2)


def _mid_body(acc_ref, htp_ref, dv_ref, m_ref, w_ref, b_ref, ht_ref, mo_ref):
    dv = dv_ref[...]
    h = jnp.maximum(dv * (acc_ref[0] + acc_ref[1] + htp_ref[...]), 0.0)
    mo_ref[...] = jnp.maximum(m_ref[...], h)
    ht_ref[...] = (jnp.dot(h, w_ref[...],
                           preferred_element_type=jnp.float32) + b_ref[...]) * dv


def _mid_call(acc, htp, dv, m, Wt, bt):
    return pl.pallas_call(
        _mid_body,
        out_shape=[jax.ShapeDtypeStruct((NP8, 128), jnp.float32),
                   jax.ShapeDtypeStruct((NP8, 128), jnp.float32)],
    )(acc, htp, dv, m, Wt, bt)


def _final_body(acc_ref, htp_ref, dv_ref, m_ref, w_ref, b_ref, red_ref, bc_ref,
                out_ref):
    h = jnp.maximum(dv_ref[...] * (acc_ref[0] + acc_ref[1] + htp_ref[...]), 0.0)
    m = jnp.maximum(m_ref[...], h)
    o = jnp.dot(m, w_ref[...], preferred_element_type=jnp.float32) + b_ref[...]
    o = o - jnp.max(o)
    eo = jnp.exp(o)
    s = jnp.dot(eo, red_ref[...], preferred_element_type=jnp.float32)
    lse = jnp.dot(jnp.log(s), bc_ref[...],
                  preferred_element_type=jnp.float32)
    out_ref[...] = o - lse


def _final_call(acc, htp, dv, m, fcWt, fcbt, red, bc):
    return pl.pallas_call(
        _final_body,
        out_shape=jax.ShapeDtypeStruct((NP8, 8 * NCLS), jnp.float32),
    )(acc, htp, dv, m, fcWt, fcbt, red, bc)


def kernel(x, edge_index, W0, b0, W1, b1, W2, b2, W3, b3, W4, b4, W5, b5,
           fcW, fcb):
    f32 = jnp.float32
    eye8 = jnp.eye(8, dtype=f32)
    pad = (jnp.arange(EPAD - E, dtype=jnp.int32) % (NPAD - N)) + N
    srcp = jnp.concatenate([edge_index[0], pad]).reshape(NW, CPW, CHUNK)
    dstp = jnp.concatenate([edge_index[1], pad]).reshape(NW, CPW, CHUNK)
    xp = jnp.pad(x, ((0, NPAD - N), (0, 0))).reshape(NP8, 8 * 128)
    zeros1 = jnp.zeros((NPAD,), f32)
    ones1 = jnp.ones((NPAD,), f32)
    W0t = jnp.kron(eye8, W0)
    b0t = jnp.tile(b0, 8).reshape(1, 128)
    fcWt = jnp.kron(eye8, fcW)
    fcbt = jnp.tile(fcb, 8).reshape(1, 8 * NCLS)
    red = jnp.kron(eye8, jnp.ones((NCLS, 1), f32))
    bc = jnp.kron(eye8, jnp.ones((1, NCLS), f32))

    deg2 = _deg_call()(dstp, ones1, zeros1).reshape(NC, NP8, 128)
    ht, dv = _prep_call(xp, W0t, b0t, deg2)
    m = jnp.zeros((NP8, 128), f32)
    Wts = [(jnp.kron(eye8, W), jnp.tile(b, 8).reshape(1, 128))
           for W, b in ((W1, b1), (W2, b2), (W3, b3), (W4, b4), (W5, b5))]
    for Wt, bt in Wts:
        acc = _edge_call()(ht.reshape(NPAD, D), srcp, dstp)
        ht, m = _mid_call(acc.reshape(NC, NP8, 128), ht, dv, m, Wt, bt)
    acc = _edge_call()(ht.reshape(NPAD, D), srcp, dstp)
    out = _final_call(acc.reshape(NC, NP8, 128), ht, dv, m, fcWt, fcbt, red, bc)
    return out.reshape(NPAD, NCLS)[:N]
